# baseline (speedup 1.0000x reference)
import functools

import jax
import jax.numpy as jnp
from jax import lax
from jax.experimental import pallas as pl
from jax.experimental.pallas import tpu as pltpu
from jax.experimental.pallas import tpu_sc as plsc

VOCAB = 51200
DIM = 2048
BATCH = 4
SEQ = 8192
TOKENS = BATCH * SEQ

NC = 2
NS = 16
NW = NC * NS
B_PER_W = TOKENS // NW
CHUNK = 16
NCHUNK = B_PER_W // CHUNK
NBUF = 3

_MESH = plsc.VectorSubcoreMesh(core_axis_name="c", subcore_axis_name="s")


@functools.partial(
    pl.kernel,
    out_type=jax.ShapeDtypeStruct((TOKENS, DIM), jnp.float32),
    mesh=_MESH,
    scratch_types=[
        pltpu.VMEM((NCHUNK + 1, CHUNK), jnp.int32),
        pltpu.VMEM((NBUF, CHUNK, DIM), jnp.float32),
        pltpu.SemaphoreType.DMA,
        pltpu.SemaphoreType.DMA,
        pltpu.SemaphoreType.DMA,
        pltpu.SemaphoreType.DMA,
        pltpu.SemaphoreType.DMA,
        pltpu.SemaphoreType.DMA,
    ],
)
def _embed_sc(idx_hbm, table_hbm, out_hbm, idx_v, bufs,
              g0, g1, g2, w0, w1, w2):
    gsem = (g0, g1, g2)
    wsem = (w0, w1, w2)
    wid = lax.axis_index("s") * NC + lax.axis_index("c")
    base = wid * B_PER_W
    pltpu.sync_copy(idx_hbm.at[wid], idx_v.at[pl.ds(0, NCHUNK)])
    idx_v[NCHUNK] = jnp.zeros((CHUNK,), jnp.int32)

    def fire_g(c, b):
        pltpu.async_copy(table_hbm.at[idx_v.at[c]], bufs.at[b], gsem[b])

    def wait_g(c, b):
        pltpu.make_async_copy(
            table_hbm.at[idx_v.at[c]], bufs.at[b], gsem[b]).wait()

    def fire_w(c, b):
        pltpu.async_copy(
            bufs.at[b], out_hbm.at[pl.ds(base + c * CHUNK, CHUNK)], wsem[b])

    def wait_w(c, b):
        pltpu.make_async_copy(
            bufs.at[b], out_hbm.at[pl.ds(base + c * CHUNK, CHUNK)],
            wsem[b]).wait()

    fire_g(0, 0)
    fire_g(1, 1)
    wait_g(0, 0); fire_w(0, 0); fire_g(2, 2)
    wait_g(1, 1); fire_w(1, 1); wait_w(0, 0); fire_g(3, 0)
    wait_g(2, 2); fire_w(2, 2); wait_w(1, 1); fire_g(4, 1)

    def group_body(g, carry):
        c0 = 3 * g
        wait_g(c0, 0); fire_w(c0, 0); wait_w(c0 - 1, 2); fire_g(c0 + 2, 2)
        wait_g(c0 + 1, 1); fire_w(c0 + 1, 1); wait_w(c0, 0); fire_g(c0 + 3, 0)
        wait_g(c0 + 2, 2); fire_w(c0 + 2, 2); wait_w(c0 + 1, 1); fire_g(c0 + 4, 1)
        return carry

    lax.fori_loop(1, NCHUNK // 3, group_body, 0)

    wait_g(NCHUNK - 1, 0)
    fire_w(NCHUNK - 1, 0)
    wait_w(NCHUNK - 2, 2)
    wait_w(NCHUNK - 1, 0)
    wait_g(NCHUNK, 1)


def kernel(input_ids, table):
    idx = input_ids.reshape(NW, NCHUNK, CHUNK).astype(jnp.int32)
    out = _embed_sc(idx, table)
    return out.reshape(BATCH, SEQ, DIM)


# speedup vs baseline: 1.0902x; 1.0902x over previous
import functools

import jax
import jax.numpy as jnp
from jax import lax
from jax.experimental import pallas as pl
from jax.experimental.pallas import tpu as pltpu
from jax.experimental.pallas import tpu_sc as plsc

VOCAB = 51200
DIM = 2048
BATCH = 4
SEQ = 8192
TOKENS = BATCH * SEQ

NC = 2
NS = 16
NW = NC * NS
B_PER_W = TOKENS // NW
CHUNK = 24
NPAIR = 21
TAIL = B_PER_W - 2 * NPAIR * CHUNK
IDX_PAD = 2 * NPAIR * CHUNK + 2 * CHUNK

_MESH = plsc.VectorSubcoreMesh(core_axis_name="c", subcore_axis_name="s")


@functools.partial(
    pl.kernel,
    out_type=jax.ShapeDtypeStruct((TOKENS, DIM), jnp.float32),
    mesh=_MESH,
    scratch_types=[
        pltpu.VMEM((IDX_PAD,), jnp.int32),
        pltpu.VMEM((CHUNK, DIM), jnp.float32),
        pltpu.VMEM((CHUNK, DIM), jnp.float32),
        pltpu.SemaphoreType.DMA,
        pltpu.SemaphoreType.DMA,
    ],
)
def _embed_sc(idx_hbm, table_hbm, out_hbm, idx_v, buf0, buf1, gsem0, gsem1):
    wid = lax.axis_index("s") * NC + lax.axis_index("c")
    base = wid * B_PER_W
    pltpu.sync_copy(idx_hbm.at[wid], idx_v.at[pl.ds(0, B_PER_W)])
    for p in range(B_PER_W, IDX_PAD, 16):
        idx_v[pl.ds(p, 16)] = jnp.zeros((16,), jnp.int32)

    def fire_g(c, buf, sem):
        pltpu.async_copy(
            table_hbm.at[idx_v.at[pl.ds(c * CHUNK, CHUNK)]], buf, sem)

    def wait_g(c, buf, sem):
        pltpu.make_async_copy(
            table_hbm.at[idx_v.at[pl.ds(c * CHUNK, CHUNK)]], buf, sem).wait()

    fire_g(0, buf0, gsem0)

    def pair_body(i, carry):
        g0 = 2 * i
        g1 = g0 + 1
        fire_g(g1, buf1, gsem1)
        wait_g(g0, buf0, gsem0)
        pltpu.sync_copy(buf0, out_hbm.at[pl.ds(base + g0 * CHUNK, CHUNK)])
        fire_g(g0 + 2, buf0, gsem0)
        wait_g(g1, buf1, gsem1)
        pltpu.sync_copy(buf1, out_hbm.at[pl.ds(base + g1 * CHUNK, CHUNK)])
        return carry

    lax.fori_loop(0, NPAIR, pair_body, 0)

    wait_g(2 * NPAIR, buf0, gsem0)
    pltpu.sync_copy(
        buf0.at[pl.ds(0, TAIL)],
        out_hbm.at[pl.ds(base + 2 * NPAIR * CHUNK, TAIL)])


def kernel(input_ids, table):
    idx = input_ids.reshape(NW, B_PER_W).astype(jnp.int32)
    out = _embed_sc(idx, table)
    return out.reshape(BATCH, SEQ, DIM)
